# baseline (speedup 1.0000x reference)
import jax
import jax.numpy as jnp
from jax.experimental import pallas as pl


def _matmul_block_kernel(e_ref, x_ref, o_ref):
    o_ref[...] = jnp.dot(e_ref[...], x_ref[...],
                         preferred_element_type=jnp.float32)


def _stream_matmul_t(et, xt, bn):
    k, d = et.shape
    _, n = xt.shape
    grid = (n // bn,)
    return pl.pallas_call(
        _matmul_block_kernel,
        grid=grid,
        in_specs=[
            pl.BlockSpec((k, d), lambda i: (0, 0)),
            pl.BlockSpec((d, bn), lambda i: (0, i)),
        ],
        out_specs=pl.BlockSpec((k, bn), lambda i: (0, i)),
        out_shape=jax.ShapeDtypeStruct((k, n), jnp.float32),
    )(et, xt)


def kernel(U, V, Eu, Ev):
    Pt = _stream_matmul_t(Eu.T, U.T, 1024)
    Qt = _stream_matmul_t(Ev.T, V.T, 512)
    return (Pt.T, Qt.T)


# speedup vs baseline: 1.0889x; 1.0889x over previous
import jax
import jax.numpy as jnp
from jax.experimental import pallas as pl


def _matmul_block_kernel(e_ref, x_ref, o_ref):
    o_ref[...] = jnp.dot(e_ref[...], x_ref[...],
                         preferred_element_type=jnp.float32)


def _stream_matmul_t(et, xt, bn):
    k, d = et.shape
    _, n = xt.shape
    grid = (n // bn,)
    return pl.pallas_call(
        _matmul_block_kernel,
        grid=grid,
        in_specs=[
            pl.BlockSpec((k, d), lambda i: (0, 0)),
            pl.BlockSpec((d, bn), lambda i: (0, i)),
        ],
        out_specs=pl.BlockSpec((k, bn), lambda i: (0, i)),
        out_shape=jax.ShapeDtypeStruct((k, n), jnp.float32),
    )(et, xt)


def kernel(U, V, Eu, Ev):
    Pt = _stream_matmul_t(Eu.T, U.T, 4096)
    Qt = _stream_matmul_t(Ev.T, V.T, 2048)
    return (Pt.T, Qt.T)


# speedup vs baseline: 1.1671x; 1.0718x over previous
import jax
import jax.numpy as jnp
from jax.experimental import pallas as pl


def _fused_kernel(eut_ref, evt_ref, ut_ref, vt_ref, pt_ref, qt_ref):
    pt_ref[...] = jnp.dot(eut_ref[...], ut_ref[...],
                          preferred_element_type=jnp.float32)
    qt_ref[...] = jnp.dot(evt_ref[...], vt_ref[...],
                          preferred_element_type=jnp.float32)


def kernel(U, V, Eu, Ev):
    Ut, Vt, EuT, EvT = U.T, V.T, Eu.T, Ev.T
    d, nu = Ut.shape
    _, nv = Vt.shape
    k = EuT.shape[0]
    steps = 8
    bu = nu // steps
    bv = nv // steps
    Pt, Qt = pl.pallas_call(
        _fused_kernel,
        grid=(steps,),
        in_specs=[
            pl.BlockSpec((k, d), lambda i: (0, 0)),
            pl.BlockSpec((k, d), lambda i: (0, 0)),
            pl.BlockSpec((d, bu), lambda i: (0, i)),
            pl.BlockSpec((d, bv), lambda i: (0, i)),
        ],
        out_specs=[
            pl.BlockSpec((k, bu), lambda i: (0, i)),
            pl.BlockSpec((k, bv), lambda i: (0, i)),
        ],
        out_shape=[
            jax.ShapeDtypeStruct((k, nu), jnp.float32),
            jax.ShapeDtypeStruct((k, nv), jnp.float32),
        ],
    )(EuT, EvT, Ut, Vt)
    return (Pt.T, Qt.T)


# speedup vs baseline: 1.1893x; 1.0191x over previous
import jax
import jax.numpy as jnp
from jax.experimental import pallas as pl


def _fused_kernel(eut_ref, evt_ref, ut_ref, vt_ref, pt_ref, qt_ref):
    pt_ref[...] = jnp.dot(eut_ref[...], ut_ref[...],
                          preferred_element_type=jnp.float32)
    qt_ref[...] = jnp.dot(evt_ref[...], vt_ref[...],
                          preferred_element_type=jnp.float32)


def kernel(U, V, Eu, Ev):
    Ut, Vt, EuT, EvT = U.T, V.T, Eu.T, Ev.T
    d, nu = Ut.shape
    _, nv = Vt.shape
    k = EuT.shape[0]
    steps = 16
    bu = nu // steps
    bv = nv // steps
    Pt, Qt = pl.pallas_call(
        _fused_kernel,
        grid=(steps,),
        in_specs=[
            pl.BlockSpec((k, d), lambda i: (0, 0)),
            pl.BlockSpec((k, d), lambda i: (0, 0)),
            pl.BlockSpec((d, bu), lambda i: (0, i)),
            pl.BlockSpec((d, bv), lambda i: (0, i)),
        ],
        out_specs=[
            pl.BlockSpec((k, bu), lambda i: (0, i)),
            pl.BlockSpec((k, bv), lambda i: (0, i)),
        ],
        out_shape=[
            jax.ShapeDtypeStruct((k, nu), jnp.float32),
            jax.ShapeDtypeStruct((k, nv), jnp.float32),
        ],
    )(EuT, EvT, Ut, Vt)
    return (Pt.T, Qt.T)
